# speedup vs baseline: 1.4148x; 1.4148x over previous
import functools

import jax
import jax.numpy as jnp
from jax import lax
from jax.experimental import pallas as pl
from jax.experimental.pallas import tpu as pltpu
from jax.experimental.pallas import tpu_sc as plsc

VOCAB = 49408
CTX = 77
DIM = 512
BATCH = 4096

B = BATCH * CTX
NC, NS = 2, 16
NW = NC * NS
BPW = B // NW
CHUNK = 64
NCH = BPW // CHUNK
NBUF = 2
NGRP = NCH // NBUF

assert BPW * NW == B and NCH * CHUNK == BPW and NGRP * NBUF == NCH

_mesh = plsc.VectorSubcoreMesh(core_axis_name="c", subcore_axis_name="s")


@functools.partial(
    pl.kernel,
    out_type=jax.ShapeDtypeStruct((B, DIM), jnp.float32),
    mesh=_mesh,
    scratch_types=[
        pltpu.VMEM((NCH, CHUNK), jnp.int32),
        pltpu.VMEM((NBUF, CHUNK, DIM), jnp.float32),
        pltpu.SemaphoreType.DMA,
        pltpu.SemaphoreType.DMA,
        pltpu.SemaphoreType.DMA,
        pltpu.SemaphoreType.DMA,
    ],
)
def _embed_gather(table_hbm, idx_hbm, out_hbm, idx_v, rows_v, g0, g1, w0, w1):
    wid = lax.axis_index("s") * NC + lax.axis_index("c")
    base = wid * BPW
    gsems = (g0, g1)
    wsems = (w0, w1)

    pltpu.sync_copy(idx_hbm.at[wid], idx_v)

    def gather_desc(j, b):
        return pltpu.make_async_copy(
            table_hbm.at[idx_v.at[j]], rows_v.at[b], gsems[b])

    def write_desc(j, b):
        return pltpu.make_async_copy(
            rows_v.at[b], out_hbm.at[pl.ds(base + j * CHUNK, CHUNK)], wsems[b])

    for b in range(NBUF):
        gather_desc(b, b).start()

    def body(g, carry):
        for b in range(NBUF):
            j = g * NBUF + b
            gather_desc(j, b).wait()
            write_desc(j, b).start()
            @pl.when(g < NGRP - 1)
            def _():
                write_desc(j, b).wait()
                gather_desc(j + NBUF, b).start()
        return carry

    lax.fori_loop(0, NGRP, body, 0, unroll=False)

    for b in range(NBUF):
        write_desc(NCH - NBUF + b, b).wait()


def kernel(tokenized_prompts, token_embedding_weight):
    idx = tokenized_prompts.reshape(NW, NCH, CHUNK)
    out = _embed_gather(token_embedding_weight, idx)
    return out.reshape(BATCH, CTX, DIM), tokenized_prompts


# speedup vs baseline: 1.4177x; 1.0020x over previous
import functools

import jax
import jax.numpy as jnp
from jax import lax
from jax.experimental import pallas as pl
from jax.experimental.pallas import tpu as pltpu
from jax.experimental.pallas import tpu_sc as plsc

VOCAB = 49408
CTX = 77
DIM = 512
BATCH = 4096

B = BATCH * CTX
NC, NS = 2, 16
NW = NC * NS
BPW = B // NW
CHUNK = 112
NCH = BPW // CHUNK
NBUF = 2
NGRP = NCH // NBUF

assert BPW * NW == B and NCH * CHUNK == BPW and NGRP * NBUF == NCH

_mesh = plsc.VectorSubcoreMesh(core_axis_name="c", subcore_axis_name="s")


@functools.partial(
    pl.kernel,
    out_type=jax.ShapeDtypeStruct((B, DIM), jnp.float32),
    mesh=_mesh,
    scratch_types=[
        pltpu.VMEM((NCH, CHUNK), jnp.int32),
        pltpu.VMEM((NBUF, CHUNK, DIM), jnp.float32),
        pltpu.SemaphoreType.DMA,
        pltpu.SemaphoreType.DMA,
        pltpu.SemaphoreType.DMA,
        pltpu.SemaphoreType.DMA,
    ],
)
def _embed_gather(table_hbm, idx_hbm, out_hbm, idx_v, rows_v, g0, g1, w0, w1):
    wid = lax.axis_index("s") * NC + lax.axis_index("c")
    base = wid * BPW
    gsems = (g0, g1)
    wsems = (w0, w1)

    pltpu.sync_copy(idx_hbm.at[wid], idx_v)

    def gather_desc(j, b):
        return pltpu.make_async_copy(
            table_hbm.at[idx_v.at[j]], rows_v.at[b], gsems[b])

    def write_desc(j, b):
        return pltpu.make_async_copy(
            rows_v.at[b], out_hbm.at[pl.ds(base + j * CHUNK, CHUNK)], wsems[b])

    for b in range(NBUF):
        gather_desc(b, b).start()

    def body(g, carry):
        for b in range(NBUF):
            j = g * NBUF + b
            gather_desc(j, b).wait()
            write_desc(j, b).start()
            @pl.when(g < NGRP - 1)
            def _():
                write_desc(j, b).wait()
                gather_desc(j + NBUF, b).start()
        return carry

    lax.fori_loop(0, NGRP, body, 0, unroll=False)

    for b in range(NBUF):
        write_desc(NCH - NBUF + b, b).wait()


def kernel(tokenized_prompts, token_embedding_weight):
    idx = tokenized_prompts.reshape(NW, NCH, CHUNK)
    out = _embed_gather(token_embedding_weight, idx)
    return out.reshape(BATCH, CTX, DIM), tokenized_prompts


# speedup vs baseline: 2.3467x; 1.6553x over previous
import functools

import jax
import jax.numpy as jnp
from jax import lax
from jax.experimental import pallas as pl
from jax.experimental.pallas import tpu as pltpu
from jax.experimental.pallas import tpu_sc as plsc

VOCAB = 49408
CTX = 77
DIM = 512
BATCH = 4096

NC, NS = 2, 16
NW = NC * NS
BPW = BATCH // NW
NBUF = 2

assert BPW * NW == BATCH and BPW % NBUF == 0

_mesh = plsc.VectorSubcoreMesh(core_axis_name="c", subcore_axis_name="s")


@functools.partial(
    pl.kernel,
    out_type=jax.ShapeDtypeStruct((BATCH, CTX, DIM), jnp.float32),
    mesh=_mesh,
    scratch_types=[
        pltpu.VMEM((BPW, CTX), jnp.int32),
        pltpu.VMEM((NBUF, CTX, DIM), jnp.float32),
        pltpu.SemaphoreType.DMA,
        pltpu.SemaphoreType.DMA,
        pltpu.SemaphoreType.DMA,
        pltpu.SemaphoreType.DMA,
    ],
)
def _embed_gather(table_hbm, idx_hbm, out_hbm, idx_v, rows_v, g0, g1, w0, w1):
    wid = lax.axis_index("s") * NC + lax.axis_index("c")
    base = wid * BPW
    gsems = (g0, g1)
    wsems = (w0, w1)

    pltpu.sync_copy(idx_hbm.at[wid], idx_v)

    def gather_desc(j, b):
        return pltpu.make_async_copy(
            table_hbm.at[idx_v.at[j]], rows_v.at[b], gsems[b])

    def write_desc(j, b):
        return pltpu.make_async_copy(
            rows_v.at[b], out_hbm.at[base + j], wsems[b])

    for b in range(NBUF):
        gather_desc(b, b).start()

    ngrp = BPW // NBUF

    def body(g, carry):
        for b in range(NBUF):
            j = g * NBUF + b
            gather_desc(j, b).wait()
            write_desc(j, b).start()
            @pl.when(g < ngrp - 1)
            def _():
                write_desc(j, b).wait()
                gather_desc(j + NBUF, b).start()
        return carry

    lax.fori_loop(0, ngrp, body, 0, unroll=False)

    for b in range(NBUF):
        write_desc(BPW - NBUF + b, b).wait()


def kernel(tokenized_prompts, token_embedding_weight):
    idx = tokenized_prompts.reshape(NW, BPW, CTX)
    out = _embed_gather(token_embedding_weight, idx)
    return out, tokenized_prompts
